# Initial kernel scaffold; baseline (speedup 1.0000x reference)
import functools

import jax
import jax.numpy as jnp
from jax import lax
from jax.experimental import pallas as pl
from jax.experimental.pallas import tpu as pltpu
from jax.experimental.pallas import tpu_sc as plsc

L = 16


def _tables_body(x_ref, w_ref, b_ref, ysb_ref, xy_ref):
    y = lax.dot_general(
        x_ref[...], w_ref[...], (((1,), (0,)), ((), ())),
        precision=lax.Precision.HIGHEST,
        preferred_element_type=jnp.float32,
    )
    ysb_ref[...] = y + b_ref[...]
    xy_ref[:, : y.shape[1]] = y
    xy_ref[:, y.shape[1]:] = x_ref[...]


def _build_tables(x, W1, b1_row):
    n, d = x.shape
    h = W1.shape[1]
    return pl.pallas_call(
        _tables_body,
        out_shape=(
            jax.ShapeDtypeStruct((n, h), jnp.float32),
            jax.ShapeDtypeStruct((n, h + d), jnp.float32),
        ),
    )(x, W1, b1_row)


def _make_edge_kernel(e_pad, n_workers, chunk, d, h):
    epw = e_pad // n_workers
    n_chunks = epw // chunk
    kh = h // L
    kd = d // L

    mesh = plsc.VectorSubcoreMesh(core_axis_name="c", subcore_axis_name="s")

    @functools.partial(
        pl.kernel,
        out_type=jax.ShapeDtypeStruct((e_pad, d), jnp.float32),
        mesh=mesh,
        scratch_types=[
            pltpu.VMEM((chunk,), jnp.int32),
            pltpu.VMEM((chunk,), jnp.int32),
            pltpu.VMEM((chunk, h), jnp.float32),
            pltpu.VMEM((chunk, h + d), jnp.float32),
            pltpu.VMEM((chunk, d), jnp.float32),
            pltpu.VMEM((h,), jnp.float32),
            pltpu.VMEM((L,), jnp.float32),
            pltpu.SemaphoreType.DMA,
            pltpu.SemaphoreType.DMA,
        ],
    )
    def edge_kernel(ysb_hbm, xy_hbm, src_hbm, dst_hbm, w2_hbm, b2_hbm,
                    out_hbm, src_v, dst_v, ys_v, xy_v, out_v, w2_v, b2_v,
                    sem0, sem1):
        n_cores = 2
        wid = lax.axis_index("s") * n_cores + lax.axis_index("c")
        base = wid * epw
        pltpu.sync_copy(w2_hbm, w2_v)
        pltpu.sync_copy(b2_hbm, b2_v)
        b2 = b2_v[...]

        def chunk_body(ci, _):
            off = base + ci * chunk
            pltpu.sync_copy(src_hbm.at[pl.ds(off, chunk)], src_v)
            pltpu.sync_copy(dst_hbm.at[pl.ds(off, chunk)], dst_v)
            cp_s = pltpu.async_copy(ysb_hbm.at[src_v], ys_v, sem0)
            cp_d = pltpu.async_copy(xy_hbm.at[dst_v], xy_v, sem1)
            cp_s.wait()
            cp_d.wait()

            def edge_body(e, _):
                acc = jnp.zeros((L,), jnp.float32)
                for k in range(kh):
                    ys = ys_v[e, pl.ds(k * L, L)]
                    yd = xy_v[e, pl.ds(k * L, L)]
                    hk = jnp.maximum(ys - yd, 0.0)
                    acc = acc + hk * w2_v[pl.ds(k * L, L)]
                s = jnp.sum(acc)
                att = jnp.exp(-jnp.maximum(s + b2, 0.0))
                for k in range(kd):
                    out_v[e, pl.ds(k * L, L)] = (
                        xy_v[e, pl.ds(h + k * L, L)] * att)
                return 0

            lax.fori_loop(0, chunk, edge_body, 0)
            pltpu.sync_copy(out_v, out_hbm.at[pl.ds(off, chunk)])
            return 0

        lax.fori_loop(0, n_chunks, chunk_body, 0)

    return edge_kernel


def kernel(x, edge_index, W1, b1, W2, b2):
    n, d = x.shape
    h = W1.shape[1]
    e = edge_index.shape[1]

    src = edge_index[0].astype(jnp.int32)
    dst = edge_index[1].astype(jnp.int32)

    n_workers = 32
    chunk = 80
    step = n_workers * chunk
    e_pad = ((e + step - 1) // step) * step
    if e_pad != e:
        src = jnp.pad(src, (0, e_pad - e))
        dst = jnp.pad(dst, (0, e_pad - e))

    ysb, xy = _build_tables(x, W1, b1.reshape(1, h))
    w2 = W2[:, 0]
    b2v = jnp.broadcast_to(b2, (L,))

    out = _make_edge_kernel(e_pad, n_workers, chunk, d, h)(
        ysb, xy, src, dst, w2, b2v)
    return out[:e] if e_pad != e else out


# speedup vs baseline: 1.5578x; 1.5578x over previous
import functools

import jax
import jax.numpy as jnp
from jax import lax
from jax.experimental import pallas as pl
from jax.experimental.pallas import tpu as pltpu
from jax.experimental.pallas import tpu_sc as plsc

L = 16


def _tables_body(x_ref, w_ref, b_ref, ysb_ref, xy_ref):
    y = lax.dot_general(
        x_ref[...], w_ref[...], (((1,), (0,)), ((), ())),
        precision=lax.Precision.HIGHEST,
        preferred_element_type=jnp.float32,
    )
    ysb_ref[...] = y + b_ref[...]
    xy_ref[:, : y.shape[1]] = y
    xy_ref[:, y.shape[1]:] = x_ref[...]


def _build_tables(x, W1, b1_row):
    n, d = x.shape
    h = W1.shape[1]
    return pl.pallas_call(
        _tables_body,
        out_shape=(
            jax.ShapeDtypeStruct((n, h), jnp.float32),
            jax.ShapeDtypeStruct((n, h + d), jnp.float32),
        ),
    )(x, W1, b1_row)


def _make_edge_kernel(e_pad, n_workers, chunk, d, h):
    epw = e_pad // n_workers
    n_chunks = epw // chunk
    kh = h // L
    kd = d // L

    mesh = plsc.VectorSubcoreMesh(core_axis_name="c", subcore_axis_name="s")

    @functools.partial(
        pl.kernel,
        out_type=jax.ShapeDtypeStruct((e_pad, d), jnp.float32),
        mesh=mesh,
        scratch_types=[
            pltpu.VMEM((chunk,), jnp.int32),
            pltpu.VMEM((chunk,), jnp.int32),
            pltpu.VMEM((chunk, h), jnp.float32),
            pltpu.VMEM((chunk, h + d), jnp.float32),
            pltpu.VMEM((chunk, d), jnp.float32),
            pltpu.VMEM((h,), jnp.float32),
            pltpu.VMEM((L,), jnp.float32),
            pltpu.SemaphoreType.DMA,
            pltpu.SemaphoreType.DMA,
        ],
        compiler_params=pltpu.CompilerParams(needs_layout_passes=False),
    )
    def edge_kernel(ysb_hbm, xy_hbm, src_hbm, dst_hbm, w2_hbm, b2_hbm,
                    out_hbm, src_v, dst_v, ys_v, xy_v, out_v, w2_v, b2_v,
                    sem0, sem1):
        n_cores = 2
        wid = lax.axis_index("s") * n_cores + lax.axis_index("c")
        base = wid * epw
        pltpu.sync_copy(w2_hbm, w2_v)
        pltpu.sync_copy(b2_hbm, b2_v)
        b2 = b2_v[...]

        def chunk_body(ci, _):
            off = base + ci * chunk
            pltpu.sync_copy(src_hbm.at[pl.ds(off, chunk)], src_v)
            pltpu.sync_copy(dst_hbm.at[pl.ds(off, chunk)], dst_v)
            cp_s = pltpu.async_copy(ysb_hbm.at[src_v], ys_v, sem0)
            cp_d = pltpu.async_copy(xy_hbm.at[dst_v], xy_v, sem1)
            cp_s.wait()
            cp_d.wait()

            def edge_body(e, _):
                acc = jnp.zeros((L,), jnp.float32)
                for k in range(kh):
                    ys = ys_v[e, pl.ds(k * L, L)]
                    yd = xy_v[e, pl.ds(k * L, L)]
                    hk = jnp.maximum(ys - yd, 0.0)
                    acc = acc + hk * w2_v[pl.ds(k * L, L)]
                s = (plsc.cumsum(acc)
                     + lax.rev(plsc.cumsum(lax.rev(acc, (0,))), (0,))
                     - acc)
                att = jnp.exp(-jnp.maximum(s + b2, 0.0))
                for k in range(kd):
                    out_v[e, pl.ds(k * L, L)] = (
                        xy_v[e, pl.ds(h + k * L, L)] * att)
                return 0

            lax.fori_loop(0, chunk, edge_body, 0)
            pltpu.sync_copy(out_v, out_hbm.at[pl.ds(off, chunk)])
            return 0

        lax.fori_loop(0, n_chunks, chunk_body, 0)

    return edge_kernel


def kernel(x, edge_index, W1, b1, W2, b2):
    n, d = x.shape
    h = W1.shape[1]
    e = edge_index.shape[1]

    src = edge_index[0].astype(jnp.int32)
    dst = edge_index[1].astype(jnp.int32)

    n_workers = 32
    chunk = 80
    step = n_workers * chunk
    e_pad = ((e + step - 1) // step) * step
    if e_pad != e:
        src = jnp.pad(src, (0, e_pad - e))
        dst = jnp.pad(dst, (0, e_pad - e))

    ysb, xy = _build_tables(x, W1, b1.reshape(1, h))
    w2 = W2[:, 0]
    b2v = jnp.broadcast_to(b2, (L,))

    out = _make_edge_kernel(e_pad, n_workers, chunk, d, h)(
        ysb, xy, src, dst, w2, b2v)
    return out[:e] if e_pad != e else out


# speedup vs baseline: 5.7046x; 3.6619x over previous
import functools

import jax
import jax.numpy as jnp
from jax import lax
from jax.experimental import pallas as pl
from jax.experimental.pallas import tpu as pltpu
from jax.experimental.pallas import tpu_sc as plsc

L = 16


def _tables_body(x_ref, w_ref, b_ref, ysb_ref, xy_ref):
    y = lax.dot_general(
        x_ref[...], w_ref[...], (((1,), (0,)), ((), ())),
        precision=lax.Precision.HIGHEST,
        preferred_element_type=jnp.float32,
    )
    ysb_ref[...] = y + b_ref[...]
    xy_ref[:, : y.shape[1]] = y
    xy_ref[:, y.shape[1]:] = x_ref[...]


def _build_tables(x, W1, b1_row):
    n, d = x.shape
    h = W1.shape[1]
    return pl.pallas_call(
        _tables_body,
        out_shape=(
            jax.ShapeDtypeStruct((n, h), jnp.float32),
            jax.ShapeDtypeStruct((n, h + d), jnp.float32),
        ),
    )(x, W1, b1_row)


def _make_edge_kernel(e_pad, n_workers, chunk, d, h):
    epw = e_pad // n_workers
    n_chunks = epw // chunk
    n_pairs = n_chunks // 2
    kh = h // L
    kd = d // L

    mesh = plsc.VectorSubcoreMesh(core_axis_name="c", subcore_axis_name="s")

    @functools.partial(
        pl.kernel,
        out_type=jax.ShapeDtypeStruct((e_pad, d), jnp.float32),
        mesh=mesh,
        scratch_types=[
            pltpu.VMEM((epw,), jnp.int32),
            pltpu.VMEM((epw,), jnp.int32),
            pltpu.VMEM((2, chunk, h), jnp.float32),
            pltpu.VMEM((2, chunk, h + d), jnp.float32),
            pltpu.VMEM((2, chunk, d), jnp.float32),
            pltpu.VMEM((h,), jnp.float32),
            pltpu.VMEM((L,), jnp.float32),
            pltpu.SemaphoreType.DMA,
            pltpu.SemaphoreType.DMA,
            pltpu.SemaphoreType.DMA,
            pltpu.SemaphoreType.DMA,
        ],
        compiler_params=pltpu.CompilerParams(needs_layout_passes=False),
    )
    def edge_kernel(ysb_hbm, xy_hbm, src_hbm, dst_hbm, w2_hbm, b2_hbm,
                    out_hbm, src_v, dst_v, ys_v, xy_v, out_v, w2_v, b2_v,
                    gsem0, gsem1, osem0, osem1):
        n_cores = 2
        wid = lax.axis_index("s") * n_cores + lax.axis_index("c")
        base = wid * epw
        pltpu.sync_copy(src_hbm.at[pl.ds(base, epw)], src_v)
        pltpu.sync_copy(dst_hbm.at[pl.ds(base, epw)], dst_v)
        pltpu.sync_copy(w2_hbm, w2_v)
        pltpu.sync_copy(b2_hbm, b2_v)
        b2 = b2_v[...]
        w2s = [w2_v[pl.ds(k * L, L)] for k in range(kh)]
        gsem = (gsem0, gsem1)
        osem = (osem0, osem1)

        def issue_gathers(slot, ci):
            off = ci * chunk
            pltpu.async_copy(ysb_hbm.at[src_v.at[pl.ds(off, chunk)]],
                             ys_v.at[slot], gsem[slot])
            pltpu.async_copy(xy_hbm.at[dst_v.at[pl.ds(off, chunk)]],
                             xy_v.at[slot], gsem[slot])

        def wait_gathers(slot):
            pltpu.make_async_copy(ysb_hbm.at[pl.ds(0, chunk)],
                                  ys_v.at[slot], gsem[slot]).wait()
            pltpu.make_async_copy(xy_hbm.at[pl.ds(0, chunk)],
                                  xy_v.at[slot], gsem[slot]).wait()

        def issue_store(slot, ci):
            pltpu.async_copy(out_v.at[slot],
                             out_hbm.at[pl.ds(base + ci * chunk, chunk)],
                             osem[slot])

        def wait_store(slot):
            pltpu.make_async_copy(out_v.at[slot],
                                  out_hbm.at[pl.ds(base, chunk)],
                                  osem[slot]).wait()

        def compute(slot):
            @plsc.parallel_loop(0, chunk, unroll=2)
            def _(e):
                acc = jnp.zeros((L,), jnp.float32)
                for k in range(kh):
                    ys = ys_v[slot, e, pl.ds(k * L, L)]
                    yd = xy_v[slot, e, pl.ds(k * L, L)]
                    hk = jnp.maximum(ys - yd, 0.0)
                    acc = acc + hk * w2s[k]
                s = (plsc.cumsum(acc)
                     + lax.rev(plsc.cumsum(lax.rev(acc, (0,))), (0,))
                     - acc)
                att = jnp.exp(-jnp.maximum(s + b2, 0.0))
                for k in range(kd):
                    out_v[slot, e, pl.ds(k * L, L)] = (
                        xy_v[slot, e, pl.ds(h + k * L, L)] * att)

        issue_gathers(0, 0)

        def pair_body(g, _):
            c0 = 2 * g
            issue_gathers(1, c0 + 1)
            wait_gathers(0)

            @pl.when(g > 0)
            def _():
                wait_store(0)

            compute(0)
            issue_store(0, c0)
            issue_gathers(0, jnp.minimum(c0 + 2, n_chunks - 1))
            wait_gathers(1)

            @pl.when(g > 0)
            def _():
                wait_store(1)

            compute(1)
            issue_store(1, c0 + 1)
            return 0

        lax.fori_loop(0, n_pairs, pair_body, 0)
        wait_gathers(0)
        wait_store(0)
        wait_store(1)

    return edge_kernel


def kernel(x, edge_index, W1, b1, W2, b2):
    n, d = x.shape
    h = W1.shape[1]
    e = edge_index.shape[1]

    src = edge_index[0].astype(jnp.int32)
    dst = edge_index[1].astype(jnp.int32)

    n_workers = 32
    chunk = 40
    step = n_workers * chunk * 2
    e_pad = ((e + step - 1) // step) * step
    if e_pad != e:
        src = jnp.pad(src, (0, e_pad - e))
        dst = jnp.pad(dst, (0, e_pad - e))

    ysb, xy = _build_tables(x, W1, b1.reshape(1, h))
    w2 = W2[:, 0]
    b2v = jnp.broadcast_to(b2, (L,))

    out = _make_edge_kernel(e_pad, n_workers, chunk, d, h)(
        ysb, xy, src, dst, w2, b2v)
    return out[:e] if e_pad != e else out
